# baseline (speedup 1.0000x reference)
import functools

import jax
import jax.numpy as jnp
from jax import lax
from jax.experimental import pallas as pl
from jax.experimental.pallas import tpu as pltpu
from jax.experimental.pallas import tpu_sc as plsc

N = 100000
E = 1600000
H = 32
BLK = 2000
GRID = N // BLK
PDGS = (1, 2, 11, 13, 22, 130, 211)


def _elu(x):
    return jnp.where(x > 0, x, jnp.exp(jnp.minimum(x, 0.0)) - 1.0)


def _tc1_body(xc_ref, xcat_ref, ec_ref, ep_ref, wc_ref, bc_ref, wk_ref,
              bk_ref, we_ref, be_ref, emb_ref, sum_ref, sq_ref):
    xc = xc_ref[...]
    xcat = xcat_ref[...]
    ci = xcat[:, 1] + 1
    oh_c = (ci[:, None] == jax.lax.broadcasted_iota(jnp.int32, (1, 3), 1))
    emb_c = oh_c.astype(jnp.float32) @ ec_ref[...]
    pv = jnp.abs(xcat[:, 0])
    for i, p in enumerate(PDGS):
        pv = jnp.where(pv == p, jnp.full_like(pv, i), pv)
    pv = jnp.clip(pv, 0, 6)
    oh_p = (pv[:, None] == jax.lax.broadcasted_iota(jnp.int32, (1, 7), 1))
    emb_p = oh_p.astype(jnp.float32) @ ep_ref[...]
    ecat = jnp.concatenate([emb_c, emb_p], axis=1)
    ecat = _elu(ecat @ wk_ref[...] + bk_ref[...])
    econt = _elu(xc @ wc_ref[...] + bc_ref[...])
    emb = jnp.concatenate([ecat, econt], axis=1)
    emb = _elu(emb @ we_ref[...] + be_ref[...])
    emb_ref[...] = emb
    sum_ref[...] = jnp.sum(emb, axis=0)[None, None, :]
    sq_ref[...] = jnp.sum(emb * emb, axis=0)[None, None, :]


def _tc1(x_cont, x_cat, emb_charge, emb_pdgid, W_cont, b_cont, W_cat, b_cat,
         W_enc, b_enc):
    wspec = lambda a: pl.BlockSpec(a.shape, lambda i: (0,) * a.ndim)
    return pl.pallas_call(
        _tc1_body,
        grid=(GRID,),
        in_specs=[
            pl.BlockSpec((BLK, 6), lambda i: (i, 0)),
            pl.BlockSpec((BLK, 2), lambda i: (i, 0)),
            wspec(emb_charge), wspec(emb_pdgid), wspec(W_cont),
            pl.BlockSpec((H // 2,), lambda i: (0,)),
            wspec(W_cat),
            pl.BlockSpec((H // 2,), lambda i: (0,)),
            wspec(W_enc),
            pl.BlockSpec((H,), lambda i: (0,)),
        ],
        out_specs=[
            pl.BlockSpec((BLK, H), lambda i: (i, 0)),
            pl.BlockSpec((1, 1, H), lambda i: (i, 0, 0)),
            pl.BlockSpec((1, 1, H), lambda i: (i, 0, 0)),
        ],
        out_shape=[
            jax.ShapeDtypeStruct((N, H), jnp.float32),
            jax.ShapeDtypeStruct((GRID, 1, H), jnp.float32),
            jax.ShapeDtypeStruct((GRID, 1, H), jnp.float32),
        ],
    )(x_cont, x_cat, emb_charge, emb_pdgid, W_cont, b_cont, W_cat, b_cat,
      W_enc, b_enc)


def _mm_body(x_ref, w_ref, b_ref, o_ref):
    o_ref[...] = x_ref[...] @ w_ref[...] + b_ref[...]


def _tc2(emb, Wb, bb):
    return pl.pallas_call(
        _mm_body,
        grid=(GRID,),
        in_specs=[
            pl.BlockSpec((BLK, H), lambda i: (i, 0)),
            pl.BlockSpec((H, H), lambda i: (0, 0)),
            pl.BlockSpec((H,), lambda i: (0,)),
        ],
        out_specs=pl.BlockSpec((BLK, H), lambda i: (i, 0)),
        out_shape=jax.ShapeDtypeStruct((N, H), jnp.float32),
    )(emb, Wb, bb)


def _tc3_body(emb_ref, mx_ref, wa_ref, ba_ref, agg_ref, sum_ref, sq_ref):
    a = emb_ref[...] @ wa_ref[...] + ba_ref[...]
    agg = a + mx_ref[...]
    agg = jnp.where(jnp.isfinite(agg), agg, 0.0)
    agg_ref[...] = agg
    sum_ref[...] = jnp.sum(agg, axis=0)[None, None, :]
    sq_ref[...] = jnp.sum(agg * agg, axis=0)[None, None, :]


def _tc3(emb, mx, Wa, ba):
    return pl.pallas_call(
        _tc3_body,
        grid=(GRID,),
        in_specs=[
            pl.BlockSpec((BLK, H), lambda i: (i, 0)),
            pl.BlockSpec((BLK, H), lambda i: (i, 0)),
            pl.BlockSpec((H, H), lambda i: (0, 0)),
            pl.BlockSpec((H,), lambda i: (0,)),
        ],
        out_specs=[
            pl.BlockSpec((BLK, H), lambda i: (i, 0)),
            pl.BlockSpec((1, 1, H), lambda i: (i, 0, 0)),
            pl.BlockSpec((1, 1, H), lambda i: (i, 0, 0)),
        ],
        out_shape=[
            jax.ShapeDtypeStruct((N, H), jnp.float32),
            jax.ShapeDtypeStruct((GRID, 1, H), jnp.float32),
            jax.ShapeDtypeStruct((GRID, 1, H), jnp.float32),
        ],
    )(emb, mx, Wa, ba)


def _tc4_body(emb_ref, agg_ref, s1_ref, t1_ref, s2_ref, t2_ref, w1_ref,
              b1_ref, w2_ref, b2_ref, o_ref):
    embn = emb_ref[...] * s1_ref[...] + t1_ref[...]
    aggn = agg_ref[...] * s2_ref[...] + t2_ref[...]
    h = _elu((embn + aggn) @ w1_ref[...] + b1_ref[...])
    o_ref[...] = h @ w2_ref[...] + b2_ref[...]


def _tc4(emb, agg, s1, t1, s2, t2, W_o1, b_o1, W_o2, b_o2):
    return pl.pallas_call(
        _tc4_body,
        grid=(GRID,),
        in_specs=[
            pl.BlockSpec((BLK, H), lambda i: (i, 0)),
            pl.BlockSpec((BLK, H), lambda i: (i, 0)),
            pl.BlockSpec((H,), lambda i: (0,)),
            pl.BlockSpec((H,), lambda i: (0,)),
            pl.BlockSpec((H,), lambda i: (0,)),
            pl.BlockSpec((H,), lambda i: (0,)),
            pl.BlockSpec((H, H // 2), lambda i: (0, 0)),
            pl.BlockSpec((H // 2,), lambda i: (0,)),
            pl.BlockSpec((H // 2, 1), lambda i: (0, 0)),
            pl.BlockSpec((1,), lambda i: (0,)),
        ],
        out_specs=pl.BlockSpec((BLK, 1), lambda i: (i, 0)),
        out_shape=jax.ShapeDtypeStruct((N, 1), jnp.float32),
    )(emb, agg, s1, t1, s2, t2, W_o1, b_o1, W_o2, b_o2)


OWN = N // 32
CH = 2000
NCH = E // CH
PEND = CH + 32
Q = 64


def _sc_body(src_hbm, dst_hbm, btab_hbm, out_hbm, sb0, db0, sb1, db1, ps0,
             pd0, ps1, pd1, rows0, rows1, acc, claim, sem0, sem1, semg0a,
             semg0b, semg1a, semg1b):
    wid = lax.axis_index("s") * 2 + lax.axis_index("c")
    lo = wid * OWN
    neg = jnp.full((16,), -jnp.inf, dtype=jnp.float32)
    lane = lax.iota(jnp.int32, 16)

    def init_acc(j, carry):
        acc[pl.ds(j * 16, 16)] = neg
        return carry

    lax.fori_loop(0, OWN * H // 16, init_acc, 0)

    zero16 = jnp.zeros((16,), dtype=jnp.int32)

    def init_ps(j, carry):
        ps0[pl.ds(j * 16, 16)] = zero16
        ps1[pl.ds(j * 16, 16)] = zero16
        return carry

    lax.fori_loop(0, PEND // 16, init_ps, 0)

    def fire(c, sb, db, sem):
        pltpu.async_copy(src_hbm.at[pl.ds(c * CH, CH)], sb, sem)
        pltpu.async_copy(dst_hbm.at[pl.ds(c * CH, CH)], db, sem)

    def drain(sb, db, sem):
        pltpu.make_async_copy(src_hbm.at[pl.ds(0, CH)], sb, sem).wait()
        pltpu.make_async_copy(dst_hbm.at[pl.ds(0, CH)], db, sem).wait()

    NG = CH // 16
    NGA = (NG + 1) // 2
    PH = PEND // 2

    def scan(sb, db, ps, pd):
        def scan_g(gi, carry):
            ca, cb = carry
            ga = gi
            gb = NGA + jnp.minimum(gi, NG - NGA - 1)
            da = db[pl.ds(ga * 16, 16)]
            sa = sb[pl.ds(ga * 16, 16)]
            dd = db[pl.ds(gb * 16, 16)]
            ss = sb[pl.ds(gb * 16, 16)]
            ma = (da >= lo) & (da < lo + OWN)
            mb = (dd >= lo) & (dd < lo + OWN)
            pa = plsc.all_reduce_population_count(ma)[0]
            pb = plsc.all_reduce_population_count(mb)[0]
            plsc.store_compressed(ps.at[pl.ds(ca, 16)], sa, mask=ma)
            plsc.store_compressed(pd.at[pl.ds(ca, 16)], da - lo, mask=ma)
            plsc.store_compressed(ps.at[pl.ds(PH + cb, 16)], ss, mask=mb)
            plsc.store_compressed(pd.at[pl.ds(PH + cb, 16)], dd - lo,
                                  mask=mb)
            return (ca + pa, cb + pb)

        return lax.fori_loop(0, NGA, scan_g, (0, 0), unroll=2)

    def fire_gather(ps, rows, semg, q, pbase, rbase):
        pltpu.async_copy(btab_hbm.at[ps.at[pl.ds(pbase + q * Q, Q)]],
                         rows.at[pl.ds(rbase, Q), :], semg)

    def wait_gather(ps, rows, semg, rbase):
        pltpu.make_async_copy(btab_hbm.at[ps.at[pl.ds(0, Q)]],
                              rows.at[pl.ds(rbase, Q), :], semg).wait()

    def do_quantum(cnt, qbase, pd, rows, pbase, rbase):
        nq_rows = jnp.minimum(cnt - qbase, Q)

        def upd_group(gi, c2):
            dloc = pd[pl.ds(pbase + qbase + gi * 16, 16)]
            valid = jnp.where(lane < nq_rows - gi * 16, 1, 0)
            rowi = rbase + gi * 16 + lane
            dbase = dloc * H

            def claim_round(rem):
                rm = rem > 0
                plsc.store_scatter(claim, [dloc], lane, mask=rm)
                got = plsc.load_gather(claim, [dloc], mask=rm)
                won = rm & (got == lane)
                for f in range(H):
                    rot = (lane + f) & (H - 1)
                    av = plsc.load_gather(acc, [dbase + rot], mask=won)
                    rv = plsc.load_gather(rows, [rowi, rot], mask=won)
                    plsc.store_scatter(
                        acc, [dbase + rot], jnp.maximum(av, rv), mask=won)
                return jnp.where(won, 0, rem)

            lax.while_loop(lambda r: jnp.max(r) > 0, claim_round, valid)
            return c2

        lax.fori_loop(0, (nq_rows + 15) // 16, upd_group, 0)

    def update_chain(cnt, ps, pd, rows, semg, pbase, rbase):
        wait_gather(ps, rows, semg, rbase)
        do_quantum(cnt, 0, pd, rows, pbase, rbase)

        def extra(q, carry):
            fire_gather(ps, rows, semg, q, pbase, rbase)
            wait_gather(ps, rows, semg, rbase)
            do_quantum(cnt, q * Q, pd, rows, pbase, rbase)
            return carry

        lax.fori_loop(1, (cnt + Q - 1) // Q, extra, 0)

    def fire_gathers(cab, ps, rows, semga, semgb):
        fire_gather(ps, rows, semga, 0, 0, 0)
        fire_gather(ps, rows, semgb, 0, PH, Q)

    def update(cab, ps, pd, rows, semga, semgb):
        ca, cb = cab
        update_chain(ca, ps, pd, rows, semga, 0, 0)
        update_chain(cb, ps, pd, rows, semgb, PH, Q)

    fire(0, sb0, db0, sem0)
    drain(sb0, db0, sem0)
    fire(1, sb1, db1, sem1)
    cab0 = scan(sb0, db0, ps0, pd0)
    fire_gathers(cab0, ps0, rows0, semg0a, semg0b)

    def pair_body(k, carry):
        cab0 = carry
        i = 2 * k + 1
        fire(i + 1, sb0, db0, sem0)
        drain(sb1, db1, sem1)
        cab1 = scan(sb1, db1, ps1, pd1)
        fire_gathers(cab1, ps1, rows1, semg1a, semg1b)
        update(cab0, ps0, pd0, rows0, semg0a, semg0b)
        fire(jnp.minimum(i + 2, NCH - 1), sb1, db1, sem1)
        drain(sb0, db0, sem0)
        cab0 = scan(sb0, db0, ps0, pd0)
        fire_gathers(cab0, ps0, rows0, semg0a, semg0b)
        update(cab1, ps1, pd1, rows1, semg1a, semg1b)
        return cab0

    cab0 = lax.fori_loop(0, (NCH - 2) // 2, pair_body, cab0)
    drain(sb1, db1, sem1)
    cab1 = scan(sb1, db1, ps1, pd1)
    fire_gathers(cab1, ps1, rows1, semg1a, semg1b)
    update(cab0, ps0, pd0, rows0, semg0a, semg0b)
    update(cab1, ps1, pd1, rows1, semg1a, semg1b)
    pltpu.sync_copy(acc, out_hbm.at[pl.ds(lo * H, OWN * H)])


def _segment_max(src, dst, btab):
    mesh = plsc.VectorSubcoreMesh(core_axis_name="c", subcore_axis_name="s")
    f = functools.partial(
        pl.kernel,
        mesh=mesh,
        compiler_params=pltpu.CompilerParams(
            use_tc_tiling_on_sc=False, needs_layout_passes=False),
        out_type=jax.ShapeDtypeStruct((N * H,), jnp.float32),
        scratch_types=[
            pltpu.VMEM((CH,), jnp.int32),
            pltpu.VMEM((CH,), jnp.int32),
            pltpu.VMEM((CH,), jnp.int32),
            pltpu.VMEM((CH,), jnp.int32),
            pltpu.VMEM((PEND,), jnp.int32),
            pltpu.VMEM((PEND,), jnp.int32),
            pltpu.VMEM((PEND,), jnp.int32),
            pltpu.VMEM((PEND,), jnp.int32),
            pltpu.VMEM((2 * Q, H), jnp.float32),
            pltpu.VMEM((2 * Q, H), jnp.float32),
            pltpu.VMEM((OWN * H,), jnp.float32),
            pltpu.VMEM((OWN,), jnp.int32),
            pltpu.SemaphoreType.DMA,
            pltpu.SemaphoreType.DMA,
            pltpu.SemaphoreType.DMA,
            pltpu.SemaphoreType.DMA,
            pltpu.SemaphoreType.DMA,
            pltpu.SemaphoreType.DMA,
        ],
    )(_sc_body)
    return f(src, dst, btab).reshape(N, H)


def kernel(x_cont, x_cat, edge_index, batch, emb_charge, emb_pdgid, W_cont,
           b_cont, W_cat, b_cat, W_enc, b_enc, g_all, bt_all, W_conv, b_conv,
           g_conv, bt_conv, W_o1, b_o1, W_o2, b_o2):
    emb, psum, psq = _tc1(x_cont, x_cat, emb_charge, emb_pdgid, W_cont,
                          b_cont, W_cat, b_cat, W_enc, b_enc)
    mean = jnp.sum(psum, axis=(0, 1)) / N
    var = jnp.sum(psq, axis=(0, 1)) / N - mean * mean
    s1 = g_all / jnp.sqrt(var + 1e-5)
    t1 = bt_all - mean * s1

    W1 = W_conv[:H]
    W2 = W_conv[H:]
    Wa = s1[:, None] * (W1 - W2)
    ba = t1 @ (W1 - W2) + b_conv
    Wb = s1[:, None] * W2
    bb = t1 @ W2

    btab = _tc2(emb, Wb, bb)
    mx = _segment_max(edge_index[0], edge_index[1], btab)
    agg, p2sum, p2sq = _tc3(emb, mx, Wa, ba)
    mean2 = jnp.sum(p2sum, axis=(0, 1)) / N
    var2 = jnp.sum(p2sq, axis=(0, 1)) / N - mean2 * mean2
    s2 = g_conv / jnp.sqrt(var2 + 1e-5)
    t2 = bt_conv - mean2 * s2

    out = _tc4(emb, agg, s1, t1, s2, t2, W_o1, b_o1, W_o2, b_o2)
    return jnp.squeeze(out, -1)


# speedup vs baseline: 3.2590x; 3.2590x over previous
import functools

import jax
import jax.numpy as jnp
from jax import lax
from jax.experimental import pallas as pl
from jax.experimental.pallas import tpu as pltpu
from jax.experimental.pallas import tpu_sc as plsc

N = 100000
E = 1600000
H = 32
BLK = 2000
GRID = N // BLK
PDGS = (1, 2, 11, 13, 22, 130, 211)


def _elu(x):
    return jnp.where(x > 0, x, jnp.exp(jnp.minimum(x, 0.0)) - 1.0)


def _tc1_body(xc_ref, xcat_ref, ec_ref, ep_ref, wc_ref, bc_ref, wk_ref,
              bk_ref, we_ref, be_ref, emb_ref, sum_ref, sq_ref):
    xc = xc_ref[...]
    xcat = xcat_ref[...]
    ci = xcat[:, 1] + 1
    oh_c = (ci[:, None] == jax.lax.broadcasted_iota(jnp.int32, (1, 3), 1))
    emb_c = oh_c.astype(jnp.float32) @ ec_ref[...]
    pv = jnp.abs(xcat[:, 0])
    for i, p in enumerate(PDGS):
        pv = jnp.where(pv == p, jnp.full_like(pv, i), pv)
    pv = jnp.clip(pv, 0, 6)
    oh_p = (pv[:, None] == jax.lax.broadcasted_iota(jnp.int32, (1, 7), 1))
    emb_p = oh_p.astype(jnp.float32) @ ep_ref[...]
    ecat = jnp.concatenate([emb_c, emb_p], axis=1)
    ecat = _elu(ecat @ wk_ref[...] + bk_ref[...])
    econt = _elu(xc @ wc_ref[...] + bc_ref[...])
    emb = jnp.concatenate([ecat, econt], axis=1)
    emb = _elu(emb @ we_ref[...] + be_ref[...])
    emb_ref[...] = emb
    sum_ref[...] = jnp.sum(emb, axis=0)[None, None, :]
    sq_ref[...] = jnp.sum(emb * emb, axis=0)[None, None, :]


def _tc1(x_cont, x_cat, emb_charge, emb_pdgid, W_cont, b_cont, W_cat, b_cat,
         W_enc, b_enc):
    wspec = lambda a: pl.BlockSpec(a.shape, lambda i: (0,) * a.ndim)
    return pl.pallas_call(
        _tc1_body,
        grid=(GRID,),
        in_specs=[
            pl.BlockSpec((BLK, 6), lambda i: (i, 0)),
            pl.BlockSpec((BLK, 2), lambda i: (i, 0)),
            wspec(emb_charge), wspec(emb_pdgid), wspec(W_cont),
            pl.BlockSpec((H // 2,), lambda i: (0,)),
            wspec(W_cat),
            pl.BlockSpec((H // 2,), lambda i: (0,)),
            wspec(W_enc),
            pl.BlockSpec((H,), lambda i: (0,)),
        ],
        out_specs=[
            pl.BlockSpec((BLK, H), lambda i: (i, 0)),
            pl.BlockSpec((1, 1, H), lambda i: (i, 0, 0)),
            pl.BlockSpec((1, 1, H), lambda i: (i, 0, 0)),
        ],
        out_shape=[
            jax.ShapeDtypeStruct((N, H), jnp.float32),
            jax.ShapeDtypeStruct((GRID, 1, H), jnp.float32),
            jax.ShapeDtypeStruct((GRID, 1, H), jnp.float32),
        ],
    )(x_cont, x_cat, emb_charge, emb_pdgid, W_cont, b_cont, W_cat, b_cat,
      W_enc, b_enc)


def _mm_body(x_ref, w_ref, b_ref, o_ref):
    o_ref[...] = x_ref[...] @ w_ref[...] + b_ref[...]


def _tc2(emb, Wb, bb):
    return pl.pallas_call(
        _mm_body,
        grid=(GRID,),
        in_specs=[
            pl.BlockSpec((BLK, H), lambda i: (i, 0)),
            pl.BlockSpec((H, H), lambda i: (0, 0)),
            pl.BlockSpec((H,), lambda i: (0,)),
        ],
        out_specs=pl.BlockSpec((BLK, H), lambda i: (i, 0)),
        out_shape=jax.ShapeDtypeStruct((N, H), jnp.float32),
    )(emb, Wb, bb)


def _tc3_body(emb_ref, mx_ref, wa_ref, ba_ref, agg_ref, sum_ref, sq_ref):
    a = emb_ref[...] @ wa_ref[...] + ba_ref[...]
    agg = a + mx_ref[...]
    agg = jnp.where(jnp.isfinite(agg), agg, 0.0)
    agg_ref[...] = agg
    sum_ref[...] = jnp.sum(agg, axis=0)[None, None, :]
    sq_ref[...] = jnp.sum(agg * agg, axis=0)[None, None, :]


def _tc3(emb, mx, Wa, ba):
    return pl.pallas_call(
        _tc3_body,
        grid=(GRID,),
        in_specs=[
            pl.BlockSpec((BLK, H), lambda i: (i, 0)),
            pl.BlockSpec((BLK, H), lambda i: (i, 0)),
            pl.BlockSpec((H, H), lambda i: (0, 0)),
            pl.BlockSpec((H,), lambda i: (0,)),
        ],
        out_specs=[
            pl.BlockSpec((BLK, H), lambda i: (i, 0)),
            pl.BlockSpec((1, 1, H), lambda i: (i, 0, 0)),
            pl.BlockSpec((1, 1, H), lambda i: (i, 0, 0)),
        ],
        out_shape=[
            jax.ShapeDtypeStruct((N, H), jnp.float32),
            jax.ShapeDtypeStruct((GRID, 1, H), jnp.float32),
            jax.ShapeDtypeStruct((GRID, 1, H), jnp.float32),
        ],
    )(emb, mx, Wa, ba)


def _tc4_body(emb_ref, agg_ref, s1_ref, t1_ref, s2_ref, t2_ref, w1_ref,
              b1_ref, w2_ref, b2_ref, o_ref):
    embn = emb_ref[...] * s1_ref[...] + t1_ref[...]
    aggn = agg_ref[...] * s2_ref[...] + t2_ref[...]
    h = _elu((embn + aggn) @ w1_ref[...] + b1_ref[...])
    o_ref[...] = h @ w2_ref[...] + b2_ref[...]


def _tc4(emb, agg, s1, t1, s2, t2, W_o1, b_o1, W_o2, b_o2):
    return pl.pallas_call(
        _tc4_body,
        grid=(GRID,),
        in_specs=[
            pl.BlockSpec((BLK, H), lambda i: (i, 0)),
            pl.BlockSpec((BLK, H), lambda i: (i, 0)),
            pl.BlockSpec((H,), lambda i: (0,)),
            pl.BlockSpec((H,), lambda i: (0,)),
            pl.BlockSpec((H,), lambda i: (0,)),
            pl.BlockSpec((H,), lambda i: (0,)),
            pl.BlockSpec((H, H // 2), lambda i: (0, 0)),
            pl.BlockSpec((H // 2,), lambda i: (0,)),
            pl.BlockSpec((H // 2, 1), lambda i: (0, 0)),
            pl.BlockSpec((1,), lambda i: (0,)),
        ],
        out_specs=pl.BlockSpec((BLK, 1), lambda i: (i, 0)),
        out_shape=jax.ShapeDtypeStruct((N, 1), jnp.float32),
    )(emb, agg, s1, t1, s2, t2, W_o1, b_o1, W_o2, b_o2)


OWN = N // 32
CH = 2000
NCH = E // CH
PEND = CH + 32
Q = 64


def _sc_body(src_hbm, dst_hbm, btab_hbm, out_hbm, sb0, db0, sb1, db1, ps0,
             pd0, ps1, pd1, rows0, rows1, acc, claim, sem0, sem1, semg0,
             semg1):
    wid = lax.axis_index("s") * 2 + lax.axis_index("c")
    lo = wid * OWN
    neg = jnp.full((16,), -jnp.inf, dtype=jnp.float32)
    lane = lax.iota(jnp.int32, 16)

    def init_acc(j, carry):
        acc[pl.ds(j * 16, 16)] = neg
        return carry

    lax.fori_loop(0, OWN * H // 16, init_acc, 0)

    zero16 = jnp.zeros((16,), dtype=jnp.int32)

    def init_ps(j, carry):
        ps0[pl.ds(j * 16, 16)] = zero16
        ps1[pl.ds(j * 16, 16)] = zero16
        return carry

    lax.fori_loop(0, PEND // 16, init_ps, 0)

    def fire(c, sb, db, sem):
        pltpu.async_copy(src_hbm.at[pl.ds(c * CH, CH)], sb, sem)
        pltpu.async_copy(dst_hbm.at[pl.ds(c * CH, CH)], db, sem)

    def drain(sb, db, sem):
        pltpu.make_async_copy(src_hbm.at[pl.ds(0, CH)], sb, sem).wait()
        pltpu.make_async_copy(dst_hbm.at[pl.ds(0, CH)], db, sem).wait()

    def scan(sb, db, ps, pd):
        def scan_g(g, cnt):
            d = db[pl.ds(g * 16, 16)]
            s = sb[pl.ds(g * 16, 16)]
            m = (d >= lo) & (d < lo + OWN)
            pc = plsc.all_reduce_population_count(m)[0]
            plsc.store_compressed(ps.at[pl.ds(cnt, 16)], s, mask=m)
            plsc.store_compressed(pd.at[pl.ds(cnt, 16)], d - lo, mask=m)
            return cnt + pc

        return lax.fori_loop(0, CH // 16, scan_g, 0, unroll=4)

    HQ = Q // 2

    def fire_gather(ps, rows, semg, q):
        pltpu.async_copy(btab_hbm.at[ps.at[pl.ds(q * Q, HQ)]],
                         rows.at[pl.ds(0, HQ), :], semg)
        pltpu.async_copy(btab_hbm.at[ps.at[pl.ds(q * Q + HQ, HQ)]],
                         rows.at[pl.ds(HQ, HQ), :], semg)

    def wait_gather(ps, rows, semg):
        pltpu.make_async_copy(btab_hbm.at[ps.at[pl.ds(0, HQ)]],
                              rows.at[pl.ds(0, HQ), :], semg).wait()
        pltpu.make_async_copy(btab_hbm.at[ps.at[pl.ds(0, HQ)]],
                              rows.at[pl.ds(HQ, HQ), :], semg).wait()

    def do_quantum(cnt, qbase, pd, rows, qsize):
        nq_rows = jnp.minimum(cnt - qbase, qsize)

        def upd_group(gi, c2):
            dloc = pd[pl.ds(qbase + gi * 16, 16)]
            valid = jnp.where(lane < nq_rows - gi * 16, 1, 0)
            rowi = gi * 16 + lane
            dbase = dloc * H

            def claim_round(rem):
                rm = rem > 0
                plsc.store_scatter(claim, [dloc], lane, mask=rm)
                got = plsc.load_gather(claim, [dloc], mask=rm)
                won = rm & (got == lane)
                for f in range(H):
                    rot = (lane + f) & (H - 1)
                    av = plsc.load_gather(acc, [dbase + rot], mask=won)
                    rv = plsc.load_gather(rows, [rowi, rot], mask=won)
                    plsc.store_scatter(
                        acc, [dbase + rot], jnp.maximum(av, rv), mask=won)
                return jnp.where(won, 0, rem)

            lax.while_loop(lambda r: jnp.max(r) > 0, claim_round, valid)
            return c2

        lax.fori_loop(0, (nq_rows + 15) // 16, upd_group, 0)

    def update(cnt, ps, pd, rows, semg):
        wait_gather(ps, rows, semg)
        do_quantum(cnt, 0, pd, rows, Q)

        def extra(k, carry):
            off = Q + k * 32
            pltpu.async_copy(btab_hbm.at[ps.at[pl.ds(off, 32)]],
                             rows.at[pl.ds(0, 32), :], semg)
            pltpu.make_async_copy(btab_hbm.at[ps.at[pl.ds(0, 32)]],
                                  rows.at[pl.ds(0, 32), :], semg).wait()
            do_quantum(cnt, off, pd, rows, 32)
            return carry

        lax.fori_loop(0, (jnp.maximum(cnt - Q, 0) + 31) // 32, extra, 0)

    fire(0, sb0, db0, sem0)
    drain(sb0, db0, sem0)
    fire(1, sb1, db1, sem1)
    cnt0 = scan(sb0, db0, ps0, pd0)
    fire_gather(ps0, rows0, semg0, 0)

    def pair_body(k, carry):
        cnt0, _ = carry
        i = 2 * k + 1
        fire(i + 1, sb0, db0, sem0)
        drain(sb1, db1, sem1)
        cnt1 = scan(sb1, db1, ps1, pd1)
        fire_gather(ps1, rows1, semg1, 0)
        update(cnt0, ps0, pd0, rows0, semg0)
        fire(jnp.minimum(i + 2, NCH - 1), sb1, db1, sem1)
        drain(sb0, db0, sem0)
        cnt0 = scan(sb0, db0, ps0, pd0)
        fire_gather(ps0, rows0, semg0, 0)
        update(cnt1, ps1, pd1, rows1, semg1)
        return (cnt0, 0)

    cnt0, _ = lax.fori_loop(0, (NCH - 2) // 2, pair_body, (cnt0, 0))
    drain(sb1, db1, sem1)
    cnt1 = scan(sb1, db1, ps1, pd1)
    fire_gather(ps1, rows1, semg1, 0)
    update(cnt0, ps0, pd0, rows0, semg0)
    update(cnt1, ps1, pd1, rows1, semg1)
    pltpu.sync_copy(acc, out_hbm.at[pl.ds(lo * H, OWN * H)])


def _segment_max(src, dst, btab):
    mesh = plsc.VectorSubcoreMesh(core_axis_name="c", subcore_axis_name="s")
    f = functools.partial(
        pl.kernel,
        mesh=mesh,
        compiler_params=pltpu.CompilerParams(
            use_tc_tiling_on_sc=False, needs_layout_passes=False),
        out_type=jax.ShapeDtypeStruct((N * H,), jnp.float32),
        scratch_types=[
            pltpu.VMEM((CH,), jnp.int32),
            pltpu.VMEM((CH,), jnp.int32),
            pltpu.VMEM((CH,), jnp.int32),
            pltpu.VMEM((CH,), jnp.int32),
            pltpu.VMEM((PEND,), jnp.int32),
            pltpu.VMEM((PEND,), jnp.int32),
            pltpu.VMEM((PEND,), jnp.int32),
            pltpu.VMEM((PEND,), jnp.int32),
            pltpu.VMEM((Q, H), jnp.float32),
            pltpu.VMEM((Q, H), jnp.float32),
            pltpu.VMEM((OWN * H,), jnp.float32),
            pltpu.VMEM((OWN,), jnp.int32),
            pltpu.SemaphoreType.DMA,
            pltpu.SemaphoreType.DMA,
            pltpu.SemaphoreType.DMA,
            pltpu.SemaphoreType.DMA,
        ],
    )(_sc_body)
    return f(src, dst, btab).reshape(N, H)


def kernel(x_cont, x_cat, edge_index, batch, emb_charge, emb_pdgid, W_cont,
           b_cont, W_cat, b_cat, W_enc, b_enc, g_all, bt_all, W_conv, b_conv,
           g_conv, bt_conv, W_o1, b_o1, W_o2, b_o2):
    emb, psum, psq = _tc1(x_cont, x_cat, emb_charge, emb_pdgid, W_cont,
                          b_cont, W_cat, b_cat, W_enc, b_enc)
    mean = jnp.sum(psum, axis=(0, 1)) / N
    var = jnp.sum(psq, axis=(0, 1)) / N - mean * mean
    s1 = g_all / jnp.sqrt(var + 1e-5)
    t1 = bt_all - mean * s1

    W1 = W_conv[:H]
    W2 = W_conv[H:]
    Wa = s1[:, None] * (W1 - W2)
    ba = t1 @ (W1 - W2) + b_conv
    Wb = s1[:, None] * W2
    bb = t1 @ W2

    btab = _tc2(emb, Wb, bb)
    mx = _segment_max(edge_index[0], edge_index[1], btab)
    agg, p2sum, p2sq = _tc3(emb, mx, Wa, ba)
    mean2 = jnp.sum(p2sum, axis=(0, 1)) / N
    var2 = jnp.sum(p2sq, axis=(0, 1)) / N - mean2 * mean2
    s2 = g_conv / jnp.sqrt(var2 + 1e-5)
    t2 = bt_conv - mean2 * s2

    out = _tc4(emb, agg, s1, t1, s2, t2, W_o1, b_o1, W_o2, b_o2)
    return jnp.squeeze(out, -1)
